# Initial kernel scaffold; baseline (speedup 1.0000x reference)
import jax
import jax.numpy as jnp
from jax.experimental import pallas as pl


def kernel(guide_in):
    raise NotImplementedError("write your pallas kernel here")


# speedup vs baseline: 253.3300x; 253.3300x over previous
import functools

import jax
import jax.numpy as jnp
from jax import lax
from jax.experimental import pallas as pl
from jax.experimental.pallas import tpu as pltpu
from jax.experimental.pallas import tpu_sc as plsc

H = W = 224
B = 4
C = 96
NV = H * W
N_ROW = (H - 1) * W
N_COL = H * (W - 1)
E = N_ROW + N_COL
N_OUT = NV - 1
E_PAD = 131072
SORT_R = 1024
CHUNK = 2048
N_CHUNK = 49
OBUF = 1024
OUT_PAD = 50176


def _weights_body(fm_ref, av_ref, ah_ref):
    z = jnp.zeros((H, W), jnp.float32)

    def cbody(c, carry):
        sv, sh = carry
        xc = fm_ref[0, c]
        dv = jnp.concatenate([xc[1:, :], xc[:1, :]], axis=0) - xc
        dh = jnp.concatenate([xc[:, 1:], xc[:, :1]], axis=1) - xc
        return sv + dv * dv, sh + dh * dh

    def chunk(g, carry):
        sv, sh = carry
        pv, ph = lax.fori_loop(g * 32, (g + 1) * 32, cbody, (z, z))
        return sv + pv, sh + ph

    sv, sh = lax.fori_loop(0, C // 32, chunk, (z, z))
    av_ref[0] = jnp.sqrt(sv) + 1.0
    ah_ref[0] = jnp.sqrt(sh) + 1.0


def _edge_weights(guide_in):
    av, ah = pl.pallas_call(
        _weights_body,
        grid=(B,),
        in_specs=[pl.BlockSpec((1, C, H, W), lambda b: (b, 0, 0, 0))],
        out_specs=[
            pl.BlockSpec((1, H, W), lambda b: (b, 0, 0)),
            pl.BlockSpec((1, H, W), lambda b: (b, 0, 0)),
        ],
        out_shape=[
            jax.ShapeDtypeStruct((B, H, W), jnp.float32),
            jax.ShapeDtypeStruct((B, H, W), jnp.float32),
        ],
    )(guide_in)
    w_row = av[:, : H - 1, :].reshape(B, -1)
    w_col = ah[:, :, : W - 1].reshape(B, -1)
    w = jnp.concatenate([w_row, w_col], axis=1)
    return jnp.pad(w, ((0, 0), (0, E_PAD - E)), constant_values=jnp.inf)


def _cmp_less(ka, ia, kb, ib):
    return (ka < kb) | ((ka == kb) & (ia < ib))


def _sort_body(w_ref, uv_ref, k_ref, i_ref):
    row = lax.broadcasted_iota(jnp.int32, (SORT_R, 128), 0)
    col = lax.broadcasted_iota(jnp.int32, (SORT_R, 128), 1)
    k_ref[0] = w_ref[0]
    i_ref[0] = row * 128 + col

    def stage(kk, jj):
        K = k_ref[0]
        I = i_ref[0]
        up_i = jnp.where(
            kk >= 128, row & lax.shift_right_logical(kk, 7), col & kk
        )
        up_i = (up_i == 0).astype(jnp.int32)

        def row_partner(_):
            jr = lax.shift_right_logical(jj, 7)
            Ka, Ia = pltpu.roll(K, SORT_R - jr, 0), pltpu.roll(I, SORT_R - jr, 0)
            Kb, Ib = pltpu.roll(K, jr, 0), pltpu.roll(I, jr, 0)
            isl = ((row & jr) == 0).astype(jnp.int32)
            return Ka, Ia, Kb, Ib, isl

        def lane_partner(_):
            Ka, Ia = pltpu.roll(K, 128 - jj, 1), pltpu.roll(I, 128 - jj, 1)
            Kb, Ib = pltpu.roll(K, jj, 1), pltpu.roll(I, jj, 1)
            isl = ((col & jj) == 0).astype(jnp.int32)
            return Ka, Ia, Kb, Ib, isl

        Ka, Ia, Kb, Ib, isl = lax.cond(jj >= 128, row_partner, lane_partner, 0)
        low = isl == 1
        Kp = jnp.where(low, Ka, Kb)
        Ip = jnp.where(low, Ia, Ib)
        less_i = _cmp_less(K, I, Kp, Ip).astype(jnp.int32)
        take_self = ((up_i ^ isl) ^ less_i) == 1
        k_ref[0] = jnp.where(take_self, K, Kp)
        i_ref[0] = jnp.where(take_self, I, Ip)

    def level_body(a, carry):
        kk = lax.shift_left(1, a)

        def j_cond(jj):
            return jj >= 1

        def j_body(jj):
            stage(kk, jj)
            return lax.shift_right_logical(jj, 1)

        lax.while_loop(j_cond, j_body, lax.shift_right_logical(kk, 1))
        return carry

    lax.fori_loop(1, 18, level_body, 0)

    e = i_ref[0]
    is_row = e < N_ROW
    ep = jnp.where(is_row, 0, e - N_ROW)
    hf = jnp.floor(ep.astype(jnp.float32) * (1.0 / (W - 1))).astype(jnp.int32)
    r = ep - hf * (W - 1)
    hf = hf + (r >= (W - 1)).astype(jnp.int32) - (r < 0).astype(jnp.int32)
    wc = ep - hf * (W - 1)
    u = jnp.where(is_row, e, hf * W + wc)
    v = jnp.where(is_row, e + W, u + 1)
    real = e < E
    u = jnp.where(real, u, 0)
    v = jnp.where(real, v, 0)
    uv_ref[0] = jnp.bitwise_or(lax.shift_left(u, 16), v)


def _sort_edges(w_pad):
    w3 = w_pad.reshape(B, SORT_R, 128)
    uv = pl.pallas_call(
        _sort_body,
        grid=(B,),
        in_specs=[pl.BlockSpec((1, SORT_R, 128), lambda b: (b, 0, 0))],
        out_specs=pl.BlockSpec((1, SORT_R, 128), lambda b: (b, 0, 0)),
        out_shape=jax.ShapeDtypeStruct((B, SORT_R, 128), jnp.int32),
        scratch_shapes=[
            pltpu.VMEM((1, SORT_R, 128), jnp.float32),
            pltpu.VMEM((1, SORT_R, 128), jnp.int32),
        ],
    )(w3)
    return uv.reshape(B, E_PAD)[:, : N_CHUNK * CHUNK]


def _mst_body(uv_hbm, out_hbm, parent, chunk, obuf, swp):
    cidx = lax.axis_index("c")
    sidx = lax.axis_index("s")
    active = (sidx % 8) == 0
    b = cidx * 2 + sidx // 8

    @pl.when(active)
    def _():
        lanes = lax.broadcasted_iota(jnp.int32, (16,), 0)
        lane0 = lanes == 0
        perm = lanes ^ 1

        def init_body(i, carry):
            parent[pl.ds(i * 16, 16)] = lanes + i * 16
            return carry

        lax.fori_loop(0, NV // 16, init_body, 0)

        in_base = b * (N_CHUNK * CHUNK)
        out_base = b * OUT_PAD

        def edge_body(j, carry2):
            fill, off = carry2
            uvv = plsc.load_gather(chunk, [jnp.full((16,), 0, jnp.int32) + j])
            u = lax.shift_right_logical(uvv, 16)
            v = uvv & 0xFFFF
            x = jnp.where(lane0, u, v)

            px = plsc.load_gather(parent, [x])
            for _ in range(2):
                ppx = plsc.load_gather(parent, [px])
                plsc.store_scatter(parent, [x], ppx)
                x = ppx
                px = plsc.load_gather(parent, [x])

            def w_cond(st):
                xx, pxx = st
                return jnp.any(pxx != xx)

            def w_body(st):
                xx, pxx = st
                ppx = plsc.load_gather(parent, [pxx])
                plsc.store_scatter(parent, [xx], ppx)
                return (ppx, plsc.load_gather(parent, [ppx]))

            x, _ = lax.while_loop(w_cond, w_body, (x, px))

            swp[...] = x
            xsw = plsc.load_gather(swp, [perm])
            m_take = lane0 & (x != xsw)
            plsc.store_scatter(parent, [x], xsw, mask=m_take)
            plsc.store_scatter(
                obuf, [jnp.full((16,), 0, jnp.int32) + fill], uvv, mask=m_take
            )
            t = jnp.max(jnp.where(m_take, 1, 0))

            fill2 = fill + t
            flush = fill2 == OBUF

            @pl.when(flush)
            def _():
                o = pl.multiple_of(out_base + off, OBUF)
                pltpu.sync_copy(obuf, out_hbm.at[pl.ds(o, OBUF)])

            fill3 = jnp.where(flush, 0, fill2)
            off2 = jnp.where(flush, off + OBUF, off)
            return (fill3, off2)

        def chunk_body(ci, carry):
            o = pl.multiple_of(in_base + ci * CHUNK, CHUNK)
            pltpu.sync_copy(uv_hbm.at[pl.ds(o, CHUNK)], chunk)
            return lax.fori_loop(0, CHUNK, edge_body, carry)

        fill, off = lax.fori_loop(
            0, N_CHUNK, chunk_body, (jnp.int32(0), jnp.int32(0))
        )
        o = pl.multiple_of(out_base + off, OBUF)
        pltpu.sync_copy(obuf, out_hbm.at[pl.ds(o, OBUF)])


def _mst(uv_sorted):
    mesh = plsc.VectorSubcoreMesh(
        core_axis_name="c", subcore_axis_name="s", num_cores=2, num_subcores=16
    )
    run = pl.kernel(
        _mst_body,
        out_type=jax.ShapeDtypeStruct((B * OUT_PAD,), jnp.int32),
        mesh=mesh,
        scratch_types=[
            pltpu.VMEM((NV,), jnp.int32),
            pltpu.VMEM((CHUNK,), jnp.int32),
            pltpu.VMEM((OBUF,), jnp.int32),
            pltpu.VMEM((16,), jnp.int32),
        ],
        compiler_params=pltpu.CompilerParams(needs_layout_passes=False),
    )
    return run(uv_sorted.reshape(-1)).reshape(B, OUT_PAD)


@jax.jit
def kernel(guide_in):
    w_pad = _edge_weights(guide_in)
    uv_sorted = _sort_edges(w_pad)
    out = _mst(uv_sorted)[:, :N_OUT]
    u = lax.shift_right_logical(out, 16)
    v = out & 0xFFFF
    return jnp.stack([u, v], axis=-1).astype(jnp.int32)
